# baseline (speedup 1.0000x reference)
import functools

import jax
import jax.numpy as jnp
from jax import lax
from jax.experimental import pallas as pl
from jax.experimental.pallas import tpu as pltpu
from jax.experimental.pallas import tpu_sc as plsc

N = 10000
E = 320000
D = 128

NUM_CORES = 2
NUM_SUBCORES = 16
NUM_WORKERS = NUM_CORES * NUM_SUBCORES
EPW = E // NUM_WORKERS
CHUNK = 80
NCHUNKS = EPW // CHUNK
SEGS = 5
CPS = NCHUNKS // SEGS
EPS = EPW // SEGS

ROW_STRIDE = 624
ROW_SPAN = 640
ZROWS = 8

_mesh = plsc.VectorSubcoreMesh(core_axis_name="c", subcore_axis_name="s")


@functools.partial(
    pl.kernel,
    mesh=_mesh,
    compiler_params=pltpu.CompilerParams(needs_layout_passes=False),
    out_type=jax.ShapeDtypeStruct((NUM_CORES, N, D), jnp.float32),
    scratch_types=[
        pltpu.VMEM((EPS,), jnp.int32),
        pltpu.VMEM((CPS, CHUNK), jnp.int32),
        pltpu.VMEM((EPS,), jnp.float32),
        pltpu.VMEM((CHUNK, D), jnp.float32),
        pltpu.VMEM((CHUNK, D), jnp.float32),
        pltpu.VMEM((CHUNK, D), jnp.float32),
        pltpu.VMEM((CHUNK, D), jnp.float32),
        pltpu.VMEM((ZROWS, D), jnp.float32),
        pltpu.VMEM_SHARED((N, D), jnp.float32),
        pltpu.SemaphoreType.DMA,
        pltpu.SemaphoreType.DMA,
    ],
)
def _sc_aggregate(x_hbm, src_hbm, dst3_hbm, w_hbm, out_hbm,
                  src_v, dst_v, w_v, rows0, rows1, rows2, rows3, zb_v, acc,
                  gsem, ssem):
    cid = lax.axis_index("c")
    sid = lax.axis_index("s")
    wid = sid * NUM_CORES + cid

    zero16 = jnp.zeros((16,), jnp.float32)
    for r in range(ZROWS):
        for dseg in range(D // 16):
            zb_v[r, pl.ds(dseg * 16, 16)] = zero16
    row0 = sid * ROW_STRIDE
    for k in range(ROW_SPAN // ZROWS):
        pltpu.async_copy(zb_v, acc.at[pl.ds(row0 + k * ZROWS, ZROWS), :],
                         gsem)
    for k in range(ROW_SPAN // ZROWS):
        pltpu.make_async_copy(
            zb_v, acc.at[pl.ds(row0, ZROWS), :], gsem).wait()

    plsc.subcore_barrier()

    def gather(c, rows):
        pltpu.async_copy(
            x_hbm.at[src_v.at[pl.ds(pl.multiple_of(c * CHUNK, 8), CHUNK)]],
            rows, gsem)

    def scale(c, rows):
        def scale_body(rb, c2):
            w16 = w_v[pl.ds(c * CHUNK + rb * 16, 16)]
            for j in range(16):
                wb = jnp.full((16,), w16[j])
                r = rb * 16 + j
                for dseg in range(D // 16):
                    sl = pl.ds(dseg * 16, 16)
                    rows[r, sl] = rows[r, sl] * wb
            return c2

        lax.fori_loop(0, CHUNK // 16, scale_body, 0)

    def drain(sem):
        pltpu.make_async_copy(x_hbm.at[pl.ds(0, CHUNK), :], rows0, sem).wait()

    def step(c, rows, gbuf):
        drain(gsem)
        pl.when(c >= 1)(lambda: drain(ssem))
        pl.when(c <= CPS - 4)(lambda: gather(c + 3, gbuf))
        scale(c, rows)
        pltpu.async_copy(rows, acc.at[dst_v.at[c]], ssem, add=True)

    def seg_body(s, carry):
        ebase = pl.multiple_of(wid * EPW + s * EPS, 8)
        pltpu.async_copy(src_hbm.at[pl.ds(ebase, EPS)], src_v, gsem)
        pltpu.async_copy(w_hbm.at[pl.ds(ebase, EPS)], w_v, gsem)
        pltpu.async_copy(dst3_hbm.at[wid * SEGS + s], dst_v, gsem)
        pltpu.make_async_copy(src_hbm.at[pl.ds(0, EPS)], src_v, gsem).wait()
        pltpu.make_async_copy(w_hbm.at[pl.ds(0, EPS)], w_v, gsem).wait()
        pltpu.make_async_copy(dst3_hbm.at[0], dst_v, gsem).wait()

        gather(0, rows0)
        gather(1, rows1)
        gather(2, rows2)

        def quad_body(j, c2):
            c = j * 4
            step(c, rows0, rows3)
            step(c + 1, rows1, rows0)
            step(c + 2, rows2, rows1)
            step(c + 3, rows3, rows2)
            return c2

        lax.fori_loop(0, CPS // 4, quad_body, 0)
        step(CPS - 1, rows0, rows3)
        drain(ssem)
        return carry

    lax.fori_loop(0, SEGS, seg_body, 0)
    plsc.subcore_barrier()

    pltpu.sync_copy(acc.at[pl.ds(row0, ROW_SPAN), :],
                    out_hbm.at[cid, pl.ds(row0, ROW_SPAN), :])


BM = 1000


def _linear_body(p_ref, w_ref, b_ref, o_ref):
    s = p_ref[0] + p_ref[1]
    acc = lax.dot_general(s, w_ref[...], (((1,), (1,)), ((), ())),
                          preferred_element_type=jnp.float32)
    o_ref[...] = acc + b_ref[...]


_linear = pl.pallas_call(
    _linear_body,
    grid=(N // BM,),
    in_specs=[
        pl.BlockSpec((NUM_CORES, BM, D), lambda i: (0, i, 0)),
        pl.BlockSpec((D, D), lambda i: (0, 0)),
        pl.BlockSpec((1, D), lambda i: (0, 0)),
    ],
    out_specs=pl.BlockSpec((BM, D), lambda i: (i, 0)),
    out_shape=jax.ShapeDtypeStruct((N, D), jnp.float32),
)


@jax.jit
def kernel(x, edge_index, edge_weight, W, b):
    src = edge_index[0]
    dst = edge_index[1]
    dst3 = dst.reshape(NUM_WORKERS * SEGS, CPS, CHUNK)
    partials = _sc_aggregate(x, src, dst3, edge_weight)
    return _linear(partials, W, b.reshape(1, D))


# speedup vs baseline: 1.0580x; 1.0580x over previous
import functools

import jax
import jax.numpy as jnp
from jax import lax
from jax.experimental import pallas as pl
from jax.experimental.pallas import tpu as pltpu
from jax.experimental.pallas import tpu_sc as plsc

N = 10000
E = 320000
D = 128

NUM_CORES = 2
NUM_SUBCORES = 16
NUM_WORKERS = NUM_CORES * NUM_SUBCORES
EPW = E // NUM_WORKERS
CHUNK = 80
NCHUNKS = EPW // CHUNK
SEGS = 5
CPS = NCHUNKS // SEGS
EPS = EPW // SEGS

ROW_STRIDE = 624
ROW_SPAN = 640
ZROWS = 8

_mesh = plsc.VectorSubcoreMesh(core_axis_name="c", subcore_axis_name="s")


@functools.partial(
    pl.kernel,
    mesh=_mesh,
    compiler_params=pltpu.CompilerParams(needs_layout_passes=False),
    out_type=jax.ShapeDtypeStruct((NUM_CORES, N, D), jnp.float32),
    scratch_types=[
        pltpu.VMEM((EPS,), jnp.int32),
        pltpu.VMEM((CPS, CHUNK), jnp.int32),
        pltpu.VMEM((EPS,), jnp.float32),
        pltpu.VMEM((CHUNK, D), jnp.float32),
        pltpu.VMEM((CHUNK, D), jnp.float32),
        pltpu.VMEM((CHUNK, D), jnp.float32),
        pltpu.VMEM((CHUNK, D), jnp.float32),
        pltpu.VMEM((ZROWS, D), jnp.float32),
        pltpu.VMEM_SHARED((N, D), jnp.float32),
        pltpu.SemaphoreType.DMA,
        pltpu.SemaphoreType.DMA,
    ],
)
def _sc_aggregate(x_hbm, src_hbm, dst3_hbm, w_hbm, out_hbm,
                  src_v, dst_v, w_v, rows0, rows1, rows2, rows3, zb_v, acc,
                  gsem, ssem):
    cid = lax.axis_index("c")
    sid = lax.axis_index("s")
    wid = sid * NUM_CORES + cid

    zero16 = jnp.zeros((16,), jnp.float32)
    for r in range(ZROWS):
        for dseg in range(D // 16):
            zb_v[r, pl.ds(dseg * 16, 16)] = zero16
    row0 = sid * ROW_STRIDE
    for k in range(ROW_SPAN // ZROWS):
        pltpu.async_copy(zb_v, acc.at[pl.ds(row0 + k * ZROWS, ZROWS), :],
                         gsem)
    for k in range(ROW_SPAN // ZROWS):
        pltpu.make_async_copy(
            zb_v, acc.at[pl.ds(row0, ZROWS), :], gsem).wait()

    plsc.subcore_barrier()

    def gather(c, rows):
        pltpu.async_copy(
            x_hbm.at[src_v.at[pl.ds(pl.multiple_of(c * CHUNK, 8), CHUNK)]],
            rows, gsem)

    def scale(c, rows):
        def scale_body(rb, c2):
            w16 = w_v[pl.ds(c * CHUNK + rb * 16, 16)]
            for j in range(16):
                wb = jnp.full((16,), w16[j])
                r = rb * 16 + j
                for dseg in range(D // 16):
                    sl = pl.ds(dseg * 16, 16)
                    rows[r, sl] = rows[r, sl] * wb
            return c2

        lax.fori_loop(0, CHUNK // 16, scale_body, 0)

    def drain(sem):
        pltpu.make_async_copy(x_hbm.at[pl.ds(0, CHUNK), :], rows0, sem).wait()

    def step(c, rows, gbuf):
        drain(gsem)
        scale(c, rows)
        pltpu.async_copy(rows, acc.at[dst_v.at[c]], ssem, add=True)
        pl.when(c >= 1)(lambda: drain(ssem))
        pl.when(c <= CPS - 4)(lambda: gather(c + 3, gbuf))

    def seg_body(s, carry):
        ebase = pl.multiple_of(wid * EPW + s * EPS, 8)
        pltpu.async_copy(src_hbm.at[pl.ds(ebase, EPS)], src_v, gsem)
        pltpu.async_copy(w_hbm.at[pl.ds(ebase, EPS)], w_v, gsem)
        pltpu.async_copy(dst3_hbm.at[wid * SEGS + s], dst_v, gsem)
        pltpu.make_async_copy(src_hbm.at[pl.ds(0, EPS)], src_v, gsem).wait()
        pltpu.make_async_copy(w_hbm.at[pl.ds(0, EPS)], w_v, gsem).wait()
        pltpu.make_async_copy(dst3_hbm.at[0], dst_v, gsem).wait()

        gather(0, rows0)
        gather(1, rows1)
        gather(2, rows2)

        def quad_body(j, c2):
            c = j * 4
            step(c, rows0, rows3)
            step(c + 1, rows1, rows0)
            step(c + 2, rows2, rows1)
            step(c + 3, rows3, rows2)
            return c2

        lax.fori_loop(0, CPS // 4, quad_body, 0)
        step(CPS - 1, rows0, rows3)
        drain(ssem)
        return carry

    lax.fori_loop(0, SEGS, seg_body, 0)
    plsc.subcore_barrier()

    pltpu.sync_copy(acc.at[pl.ds(row0, ROW_SPAN), :],
                    out_hbm.at[cid, pl.ds(row0, ROW_SPAN), :])


BM = 1000


def _linear_body(p_ref, w_ref, b_ref, o_ref):
    s = p_ref[0] + p_ref[1]
    acc = lax.dot_general(s, w_ref[...], (((1,), (1,)), ((), ())),
                          preferred_element_type=jnp.float32)
    o_ref[...] = acc + b_ref[...]


_linear = pl.pallas_call(
    _linear_body,
    grid=(N // BM,),
    in_specs=[
        pl.BlockSpec((NUM_CORES, BM, D), lambda i: (0, i, 0)),
        pl.BlockSpec((D, D), lambda i: (0, 0)),
        pl.BlockSpec((1, D), lambda i: (0, 0)),
    ],
    out_specs=pl.BlockSpec((BM, D), lambda i: (i, 0)),
    out_shape=jax.ShapeDtypeStruct((N, D), jnp.float32),
)


@jax.jit
def kernel(x, edge_index, edge_weight, W, b):
    src = edge_index[0]
    dst = edge_index[1]
    dst3 = dst.reshape(NUM_WORKERS * SEGS, CPS, CHUNK)
    partials = _sc_aggregate(x, src, dst3, edge_weight)
    return _linear(partials, W, b.reshape(1, D))
